# baseline (speedup 1.0000x reference)
import jax
import jax.numpy as jnp
from jax import lax
from jax.experimental import pallas as pl
from jax.experimental.pallas import tpu as pltpu
from jax.experimental.pallas import tpu_sc as plsc

_N_POINTS = 16384
_N_IN = 100000
_D_FEAT = 128

_NC = 2
_NS = 16
_NW = _NC * _NS

_N_SMALL_W = 18
_PHALF = _N_POINTS // 2
_FC = 128
_NFC = _N_POINTS // _FC // _NW
_FROWS = _NFC * _FC


def _feat_body(features_hbm, idx_hbm, out_f, sem_f0, sem_f1, sem_w0, sem_w1):
    wid = lax.axis_index("s") * _NC + lax.axis_index("c")
    fbase = wid * _FROWS

    def inner(fa, fb, fidx):
        pltpu.sync_copy(idx_hbm.at[pl.ds(fbase, _FROWS)], fidx)
        fbufs = (fa, fb)
        fsems = (sem_f0, sem_f1)
        wsems = (sem_w0, sem_w1)

        def fgather(c):
            return pltpu.async_copy(
                features_hbm.at[fidx.at[pl.ds(c * _FC, _FC)]],
                fbufs[c % 2], fsems[c % 2])

        def fwrite(c):
            return pltpu.async_copy(
                fbufs[c % 2], out_f.at[pl.ds(fbase + c * _FC, _FC)],
                wsems[c % 2])

        gs = [fgather(0), fgather(1)]
        ws = []
        for c in range(_NFC):
            gs[c].wait()
            ws.append(fwrite(c))
            if c + 2 < _NFC:
                ws[c].wait()
                gs.append(fgather(c + 2))
        ws[_NFC - 2].wait()
        ws[_NFC - 1].wait()

    pl.run_scoped(inner,
                  pltpu.VMEM((_FC, _D_FEAT), jnp.float32),
                  pltpu.VMEM((_FC, _D_FEAT), jnp.float32),
                  pltpu.VMEM((_FROWS,), jnp.int32))


def _small_body(*refs):
    small_in = refs[0:9]
    idx_hbm = refs[9]
    small_out = refs[10:19]

    wid = lax.axis_index("s") * _NC + lax.axis_index("c")

    def small_job(src, dst, h):
        def inner(rowv, ibuf, obuf):
            pltpu.sync_copy(idx_hbm.at[pl.ds(h * _PHALF, _PHALF)], ibuf)
            pltpu.sync_copy(src, rowv)

            def group(g, carry):
                s = g * 16
                iv = ibuf[pl.ds(s, 16)]
                obuf[pl.ds(s, 16)] = plsc.load_gather(rowv, [iv])
                return carry

            lax.fori_loop(0, _PHALF // 16, group, 0)
            pltpu.sync_copy(obuf, dst.at[pl.ds(h * _PHALF, _PHALF)])

        pl.run_scoped(inner,
                      pltpu.VMEM((_N_IN,), jnp.float32),
                      pltpu.VMEM((_PHALF,), jnp.int32),
                      pltpu.VMEM((_PHALF,), jnp.float32))

    for w in range(_N_SMALL_W):
        @pl.when(wid == w)
        def _(src=small_in[w // 2], dst=small_out[w // 2], h=w % 2):
            small_job(src, dst, h)


@jax.jit
def _downsample(small_rows, features, idx32):
    f32 = jnp.float32
    mesh = plsc.VectorSubcoreMesh(core_axis_name="c", subcore_axis_name="s")
    params = pltpu.CompilerParams(needs_layout_passes=False)
    feat_run = pl.kernel(
        _feat_body,
        out_type=jax.ShapeDtypeStruct((_N_POINTS, _D_FEAT), f32),
        mesh=mesh,
        compiler_params=params,
        scratch_types=[
            pltpu.SemaphoreType.DMA,
            pltpu.SemaphoreType.DMA,
            pltpu.SemaphoreType.DMA,
            pltpu.SemaphoreType.DMA,
        ],
    )
    small_run = pl.kernel(
        _small_body,
        out_type=(jax.ShapeDtypeStruct((_N_POINTS,), f32),) * 9,
        mesh=mesh,
        compiler_params=params,
    )
    out_f = feat_run(features, idx32)
    small_outs = small_run(*small_rows, idx32)
    return small_outs, out_f


def kernel(coords, features, colors, normals, idx):
    idx32 = idx.astype(jnp.int32)
    small_rows = tuple(tbl[:, c]
                       for tbl in (coords, colors, normals)
                       for c in range(3))
    small_outs, out_f = _downsample(small_rows, features, idx32)
    out_c = jnp.stack(small_outs[0:3], axis=1)
    out_col = jnp.stack(small_outs[3:6], axis=1)
    out_n = jnp.stack(small_outs[6:9], axis=1)
    return (out_c, out_f, out_col, out_n)


# speedup vs baseline: 1.0125x; 1.0125x over previous
import jax
import jax.numpy as jnp
from jax import lax
from jax.experimental import pallas as pl
from jax.experimental.pallas import tpu as pltpu
from jax.experimental.pallas import tpu_sc as plsc

_N_POINTS = 16384
_N_IN = 100000
_D_FEAT = 128

_NC = 2
_NS = 16

_N_SMALL_W = 18
_FEAT_W0 = _N_SMALL_W
_N_FEAT_W = 32 - _N_SMALL_W
_FC = 128
_FCHUNKS = (10, 10) + (9,) * 12
_PHALF = _N_POINTS // 2
_FSTARTS = tuple(sum(_FCHUNKS[:i]) for i in range(_N_FEAT_W))


def _body(*refs):
    small_in = refs[0:9]
    features_hbm, idx_hbm = refs[9:11]
    small_out = refs[11:20]
    out_f = refs[20]
    sem_f0, sem_f1, sem_w0, sem_w1 = refs[21:25]

    wid = lax.axis_index("s") * _NC + lax.axis_index("c")

    def small_job(src, dst, h):
        def inner(rowv, ibuf, obuf):
            pltpu.sync_copy(idx_hbm.at[pl.ds(h * _PHALF, _PHALF)], ibuf)
            pltpu.sync_copy(src, rowv)

            def group(g, carry):
                s = g * 16
                iv = ibuf[pl.ds(s, 16)]
                obuf[pl.ds(s, 16)] = plsc.load_gather(rowv, [iv])
                return carry

            lax.fori_loop(0, _PHALF // 16, group, 0)
            pltpu.sync_copy(obuf, dst.at[pl.ds(h * _PHALF, _PHALF)])

        pl.run_scoped(inner,
                      pltpu.VMEM((_N_IN,), jnp.float32),
                      pltpu.VMEM((_PHALF,), jnp.int32),
                      pltpu.VMEM((_PHALF,), jnp.float32))

    for w in range(_N_SMALL_W):
        @pl.when(wid == w)
        def _(src=small_in[w // 2], dst=small_out[w // 2], h=w % 2):
            small_job(src, dst, h)

    def feat_job(start, nc):
        fbase = start * _FC
        frows = nc * _FC

        def inner(fa, fb, fidx):
            pltpu.sync_copy(idx_hbm.at[pl.ds(fbase, frows)], fidx)
            fbufs = (fa, fb)
            fsems = (sem_f0, sem_f1)
            wsems = (sem_w0, sem_w1)

            def fgather(c):
                return pltpu.async_copy(
                    features_hbm.at[fidx.at[pl.ds(c * _FC, _FC)]],
                    fbufs[c % 2], fsems[c % 2])

            def fwrite(c):
                return pltpu.async_copy(
                    fbufs[c % 2], out_f.at[pl.ds(fbase + c * _FC, _FC)],
                    wsems[c % 2])

            gs = [fgather(0), fgather(1)]
            ws = []
            for c in range(nc):
                gs[c].wait()
                ws.append(fwrite(c))
                if c + 2 < nc:
                    ws[c].wait()
                    gs.append(fgather(c + 2))
            ws[nc - 2].wait()
            ws[nc - 1].wait()

        pl.run_scoped(inner,
                      pltpu.VMEM((_FC, _D_FEAT), jnp.float32),
                      pltpu.VMEM((_FC, _D_FEAT), jnp.float32),
                      pltpu.VMEM((frows,), jnp.int32))

    for w in range(_N_FEAT_W):
        @pl.when(wid == _FEAT_W0 + w)
        def _(start=_FSTARTS[w], nc=_FCHUNKS[w]):
            feat_job(start, nc)


@jax.jit
def _downsample(small_rows, features, idx32):
    f32 = jnp.float32
    vec_out = jax.ShapeDtypeStruct((_N_POINTS,), f32)
    run = pl.kernel(
        _body,
        out_type=(vec_out,) * 9 + (
            jax.ShapeDtypeStruct((_N_POINTS, _D_FEAT), f32),),
        mesh=plsc.VectorSubcoreMesh(core_axis_name="c", subcore_axis_name="s"),
        compiler_params=pltpu.CompilerParams(needs_layout_passes=False),
        scratch_types=[
            pltpu.SemaphoreType.DMA,
            pltpu.SemaphoreType.DMA,
            pltpu.SemaphoreType.DMA,
            pltpu.SemaphoreType.DMA,
        ],
    )
    return run(*small_rows, features, idx32)


def kernel(coords, features, colors, normals, idx):
    idx32 = idx.astype(jnp.int32)
    small_rows = tuple(tbl[:, c]
                       for tbl in (coords, colors, normals)
                       for c in range(3))
    outs = _downsample(small_rows, features, idx32)
    small_outs, out_f = outs[0:9], outs[9]
    out_c = jnp.stack(small_outs[0:3], axis=1)
    out_col = jnp.stack(small_outs[3:6], axis=1)
    out_n = jnp.stack(small_outs[6:9], axis=1)
    return (out_c, out_f, out_col, out_n)


# speedup vs baseline: 1.0252x; 1.0126x over previous
import jax
import jax.numpy as jnp
from jax import lax
from jax.experimental import pallas as pl
from jax.experimental.pallas import tpu as pltpu
from jax.experimental.pallas import tpu_sc as plsc

_N_POINTS = 16384
_N_IN = 100000
_D_FEAT = 128

_NC = 2
_NS = 16

_N_SMALL_W = 18
_FEAT_W0 = _N_SMALL_W
_N_FEAT_W = 32 - _N_SMALL_W
_FC = 256
_FCHUNKS = (5,) * 8 + (4,) * 6
_PHALF = _N_POINTS // 2
_FSTARTS = tuple(sum(_FCHUNKS[:i]) for i in range(_N_FEAT_W))


def _body(*refs):
    small_in = refs[0:9]
    features_hbm, idx_hbm = refs[9:11]
    small_out = refs[11:20]
    out_f = refs[20]
    sem_f0, sem_f1, sem_w0, sem_w1 = refs[21:25]

    wid = lax.axis_index("s") * _NC + lax.axis_index("c")

    def small_job(src, dst, h):
        def inner(rowv, ibuf, obuf):
            pltpu.sync_copy(idx_hbm.at[pl.ds(h * _PHALF, _PHALF)], ibuf)
            pltpu.sync_copy(src, rowv)

            def group(g, carry):
                s = g * 16
                iv = ibuf[pl.ds(s, 16)]
                obuf[pl.ds(s, 16)] = plsc.load_gather(rowv, [iv])
                return carry

            lax.fori_loop(0, _PHALF // 16, group, 0)
            pltpu.sync_copy(obuf, dst.at[pl.ds(h * _PHALF, _PHALF)])

        pl.run_scoped(inner,
                      pltpu.VMEM((_N_IN,), jnp.float32),
                      pltpu.VMEM((_PHALF,), jnp.int32),
                      pltpu.VMEM((_PHALF,), jnp.float32))

    for w in range(_N_SMALL_W):
        @pl.when(wid == w)
        def _(src=small_in[w // 2], dst=small_out[w // 2], h=w % 2):
            small_job(src, dst, h)

    def feat_job(start, nc):
        fbase = start * _FC
        frows = nc * _FC

        def inner(fa, fb, fidx):
            pltpu.sync_copy(idx_hbm.at[pl.ds(fbase, frows)], fidx)
            fbufs = (fa, fb)
            fsems = (sem_f0, sem_f1)
            wsems = (sem_w0, sem_w1)

            def fgather(c):
                return pltpu.async_copy(
                    features_hbm.at[fidx.at[pl.ds(c * _FC, _FC)]],
                    fbufs[c % 2], fsems[c % 2])

            def fwrite(c):
                return pltpu.async_copy(
                    fbufs[c % 2], out_f.at[pl.ds(fbase + c * _FC, _FC)],
                    wsems[c % 2])

            gs = [fgather(0), fgather(1)]
            ws = []
            for c in range(nc):
                gs[c].wait()
                ws.append(fwrite(c))
                if c + 2 < nc:
                    ws[c].wait()
                    gs.append(fgather(c + 2))
            ws[nc - 2].wait()
            ws[nc - 1].wait()

        pl.run_scoped(inner,
                      pltpu.VMEM((_FC, _D_FEAT), jnp.float32),
                      pltpu.VMEM((_FC, _D_FEAT), jnp.float32),
                      pltpu.VMEM((frows,), jnp.int32))

    for w in range(_N_FEAT_W):
        @pl.when(wid == _FEAT_W0 + w)
        def _(start=_FSTARTS[w], nc=_FCHUNKS[w]):
            feat_job(start, nc)


@jax.jit
def _downsample(small_rows, features, idx32):
    f32 = jnp.float32
    vec_out = jax.ShapeDtypeStruct((_N_POINTS,), f32)
    run = pl.kernel(
        _body,
        out_type=(vec_out,) * 9 + (
            jax.ShapeDtypeStruct((_N_POINTS, _D_FEAT), f32),),
        mesh=plsc.VectorSubcoreMesh(core_axis_name="c", subcore_axis_name="s"),
        compiler_params=pltpu.CompilerParams(needs_layout_passes=False),
        scratch_types=[
            pltpu.SemaphoreType.DMA,
            pltpu.SemaphoreType.DMA,
            pltpu.SemaphoreType.DMA,
            pltpu.SemaphoreType.DMA,
        ],
    )
    return run(*small_rows, features, idx32)


def kernel(coords, features, colors, normals, idx):
    idx32 = idx.astype(jnp.int32)
    small_rows = tuple(tbl[:, c]
                       for tbl in (coords, colors, normals)
                       for c in range(3))
    outs = _downsample(small_rows, features, idx32)
    small_outs, out_f = outs[0:9], outs[9]
    out_c = jnp.stack(small_outs[0:3], axis=1)
    out_col = jnp.stack(small_outs[3:6], axis=1)
    out_n = jnp.stack(small_outs[6:9], axis=1)
    return (out_c, out_f, out_col, out_n)
